# baseline (speedup 1.0000x reference)
import functools

import jax
import jax.numpy as jnp
from jax import lax
from jax.experimental import pallas as pl
from jax.experimental.pallas import tpu as pltpu
from jax.experimental.pallas import tpu_sc as plsc

VOCAB = 100000
MAX_LEN = 200
EMBED = 64
BATCH = 4096

NC = 2
NS = 16
NW = NC * NS
LANES = 16

BT = BATCH // NW
BTP = BT + 8
DT = EMBED // 8
LT = MAX_LEN // 8
NBG = 4
NBT = 2


def _body(idx_hbm, tok_hbm, pos_hbm, out_hbm, idx_v, g_v, t_v, pos_v, *sems):
    isem = sems[0:NBG]
    gsem = sems[NBG:2 * NBG]
    osem = sems[2 * NBG:2 * NBG + NBT]

    w = lax.axis_index("s") * NC + lax.axis_index("c")

    pltpu.sync_copy(pos_hbm, pos_v)

    iota = lax.iota(jnp.int32, LANES)

    def fetch_idx(l, b):
        pltpu.async_copy(idx_hbm.at[l // 8, w, l % 8], idx_v.at[b], isem[b])

    def start_gather(l, b):
        pltpu.make_async_copy(idx_hbm.at[l // 8, w, l % 8], idx_v.at[b],
                              isem[b]).wait()
        pltpu.async_copy(tok_hbm.at[idx_v.at[b]], g_v.at[b], gsem[b])

    def wait_out(l, t):
        for dt in range(DT):
            pltpu.make_async_copy(t_v.at[t, pl.ds(dt * 8, 8), pl.ds(0, BT)],
                                  out_hbm.at[l, dt, w], osem[t]).wait()

    cvs = [[(16 * j) + ((iota + r) & 15) for r in range(LANES)]
           for j in range(EMBED // LANES)]

    def compute(l, b, t):
        pltpu.make_async_copy(tok_hbm.at[idx_v.at[b]], g_v.at[b],
                              gsem[b]).wait()

        lsplat = jnp.full((LANES,), l, jnp.int32)
        for j in range(EMBED // LANES):
            pos_rots = [plsc.load_gather(pos_v, [lsplat, cv])
                        for cv in cvs[j]]

            @plsc.parallel_loop(0, BT, step=LANES, unroll=2)
            def b_body(bb):
                rv = iota + bb
                for r in range(LANES):
                    cv = cvs[j][r]
                    g = plsc.load_gather(g_v.at[b], [rv, cv])
                    plsc.store_scatter(t_v.at[t], [cv, rv], g + pos_rots[r])

        for dt in range(DT):
            pltpu.async_copy(t_v.at[t, pl.ds(dt * 8, 8), pl.ds(0, BT)],
                             out_hbm.at[l, dt, w], osem[t])

    for l in range(3):
        fetch_idx(l, l)
    start_gather(0, 0)
    start_gather(1, 1)

    def step(s, carry):
        for u in range(NBG):
            l = NBG * s + u

            @pl.when(l + 3 < MAX_LEN)
            def _():
                fetch_idx(l + 3, (u + 3) % NBG)

            @pl.when(l + 2 < MAX_LEN)
            def _():
                start_gather(l + 2, (u + 2) % NBG)

            t = u % NBT

            @pl.when(l >= NBT)
            def _():
                wait_out(l - NBT, t)

            compute(l, u, t)
        return carry

    lax.fori_loop(0, MAX_LEN // NBG, step, 0)

    for t in range(NBT):
        wait_out(MAX_LEN - NBT + t, t)


def kernel(inputs, token_table, pos_table):
    idx4 = (inputs.astype(jnp.int32).T
            .reshape(LT, 8, NW, BT)
            .transpose(0, 2, 1, 3))
    mesh = plsc.VectorSubcoreMesh(core_axis_name="c", subcore_axis_name="s")
    run = functools.partial(
        pl.kernel,
        mesh=mesh,
        compiler_params=pltpu.CompilerParams(use_tc_tiling_on_sc=False,
                                             needs_layout_passes=False),
        out_type=jax.ShapeDtypeStruct((MAX_LEN, DT, NW, 8, BT), jnp.float32),
        scratch_types=[
            pltpu.VMEM((NBG, BT), jnp.int32),
            pltpu.VMEM((NBG, BT, EMBED), jnp.float32),
            pltpu.VMEM((NBT, EMBED, BTP), jnp.float32),
            pltpu.VMEM((MAX_LEN, EMBED), jnp.float32),
        ] + [pltpu.SemaphoreType.DMA] * (2 * NBG + NBT),
    )(_body)
    out = run(idx4, token_table, pos_table)
    return out.transpose(2, 4, 0, 1, 3).reshape(BATCH, MAX_LEN, EMBED)


# speedup vs baseline: 1.1791x; 1.1791x over previous
import functools

import jax
import jax.numpy as jnp
from jax import lax
from jax.experimental import pallas as pl
from jax.experimental.pallas import tpu as pltpu
from jax.experimental.pallas import tpu_sc as plsc

VOCAB = 100000
MAX_LEN = 200
EMBED = 64
BATCH = 4096

NC = 2
NS = 16
NW = NC * NS
LANES = 16

BT = BATCH // NW
BTP = BT + 8
DT = EMBED // 8
LT = MAX_LEN // 8
NBG = 4
NBT = 2


def _body(idx_hbm, tok_hbm, pos_hbm, out_hbm, idx_v, g_v, t_v, pos_v, *sems):
    isem = sems[0:NBG]
    gsem = sems[NBG:2 * NBG]
    osem = sems[2 * NBG:2 * NBG + NBT]

    w = lax.axis_index("s") * NC + lax.axis_index("c")

    pltpu.sync_copy(pos_hbm, pos_v)

    iota = lax.iota(jnp.int32, LANES)

    def fetch_idx(l, b):
        pltpu.async_copy(idx_hbm.at[l // 8, w, l % 8], idx_v.at[b], isem[b])

    def start_gather(l, b):
        pltpu.make_async_copy(idx_hbm.at[l // 8, w, l % 8], idx_v.at[b],
                              isem[b]).wait()
        pltpu.async_copy(tok_hbm.at[idx_v.at[b]], g_v.at[b], gsem[b])

    def wait_out(l, t):
        for dt in range(DT):
            pltpu.make_async_copy(t_v.at[t, pl.ds(dt * 8, 8), pl.ds(0, BT)],
                                  out_hbm.at[l, dt, w], osem[t]).wait()

    rol1 = ((iota * 2) & 15) | (iota >> 3)
    pvs = [(rol1 + s) & 15 for s in range(LANES)]
    civs = [16 * j + iota for j in range(EMBED // LANES)]

    def compute(l, b, t):
        pltpu.make_async_copy(tok_hbm.at[idx_v.at[b]], g_v.at[b],
                              gsem[b]).wait()

        for j in range(EMBED // LANES):
            civ = civs[j]
            pos_j = pos_v[l, pl.ds(16 * j, LANES)]

            @plsc.parallel_loop(0, BT, step=LANES, unroll=1)
            def b_body(bb):
                bsplat = jnp.full((LANES,), bb, jnp.int32)
                for s in range(LANES):
                    rv = bsplat + pvs[s]
                    g = plsc.load_gather(g_v.at[b], [rv, civ])
                    plsc.store_scatter(t_v.at[t], [civ, rv], g + pos_j)

        for dt in range(DT):
            pltpu.async_copy(t_v.at[t, pl.ds(dt * 8, 8), pl.ds(0, BT)],
                             out_hbm.at[l, dt, w], osem[t])

    for l in range(3):
        fetch_idx(l, l)
    start_gather(0, 0)
    start_gather(1, 1)

    def step(s, carry):
        for u in range(NBG):
            l = NBG * s + u

            @pl.when(l + 3 < MAX_LEN)
            def _():
                fetch_idx(l + 3, (u + 3) % NBG)

            @pl.when(l + 2 < MAX_LEN)
            def _():
                start_gather(l + 2, (u + 2) % NBG)

            t = u % NBT

            @pl.when(l >= NBT)
            def _():
                wait_out(l - NBT, t)

            compute(l, u, t)
        return carry

    lax.fori_loop(0, MAX_LEN // NBG, step, 0)

    for t in range(NBT):
        wait_out(MAX_LEN - NBT + t, t)


def kernel(inputs, token_table, pos_table):
    idx4 = (inputs.astype(jnp.int32).T
            .reshape(LT, 8, NW, BT)
            .transpose(0, 2, 1, 3))
    mesh = plsc.VectorSubcoreMesh(core_axis_name="c", subcore_axis_name="s")
    run = functools.partial(
        pl.kernel,
        mesh=mesh,
        compiler_params=pltpu.CompilerParams(use_tc_tiling_on_sc=False,
                                             needs_layout_passes=False),
        out_type=jax.ShapeDtypeStruct((MAX_LEN, DT, NW, 8, BT), jnp.float32),
        scratch_types=[
            pltpu.VMEM((NBG, BT), jnp.int32),
            pltpu.VMEM((NBG, BT, EMBED), jnp.float32),
            pltpu.VMEM((NBT, EMBED, BTP), jnp.float32),
            pltpu.VMEM((MAX_LEN, EMBED), jnp.float32),
        ] + [pltpu.SemaphoreType.DMA] * (2 * NBG + NBT),
    )(_body)
    out = run(idx4, token_table, pos_table)
    return out.transpose(2, 4, 0, 1, 3).reshape(BATCH, MAX_LEN, EMBED)


# speedup vs baseline: 2.5592x; 2.1704x over previous
import functools

import jax
import jax.numpy as jnp
from jax import lax
from jax.experimental import pallas as pl
from jax.experimental.pallas import tpu as pltpu
from jax.experimental.pallas import tpu_sc as plsc

VOCAB = 100000
MAX_LEN = 200
EMBED = 64
BATCH = 4096

NC = 2
NS = 16
NW = NC * NS
LANES = 16

BT = BATCH // NW
BTP = BT + 8
DT = EMBED // 8
LT = MAX_LEN // 8
NBG = 8
NBT = 4


def _body(idx_hbm, tok_hbm, pos_hbm, out_hbm, idx_v, g_v, t_v, pos_v, *sems):
    isem = sems[0:NBG]
    gsem = sems[NBG:2 * NBG]
    osem = sems[2 * NBG:2 * NBG + NBT]

    w = lax.axis_index("s") * NC + lax.axis_index("c")

    pltpu.sync_copy(pos_hbm, pos_v)

    iota = lax.iota(jnp.int32, LANES)

    def fetch_idx(l, b):
        pltpu.async_copy(idx_hbm.at[l // 8, w, l % 8], idx_v.at[b], isem[b])

    def start_gather(l, b):
        pltpu.make_async_copy(idx_hbm.at[l // 8, w, l % 8], idx_v.at[b],
                              isem[b]).wait()
        pltpu.async_copy(tok_hbm.at[idx_v.at[b]], g_v.at[b], gsem[b])

    def wait_out(l, t):
        pltpu.make_async_copy(t_v.at[t, :, :, pl.ds(0, BT)],
                              out_hbm.at[l, :, w], osem[t]).wait()

    dts = [(iota + 16 * j) // 8 for j in range(EMBED // LANES)]
    dls = [(iota + 16 * j) % 8 for j in range(EMBED // LANES)]

    def compute(l, b, t):
        pltpu.make_async_copy(tok_hbm.at[idx_v.at[b]], g_v.at[b],
                              gsem[b]).wait()

        pv = [pos_v[l, pl.ds(16 * j, LANES)] for j in range(EMBED // LANES)]

        @plsc.parallel_loop(0, BT, unroll=4)
        def b_body(bb):
            bsplat = jnp.full((LANES,), bb, jnp.int32)
            for j in range(EMBED // LANES):
                g = g_v[b, bb, pl.ds(16 * j, LANES)]
                plsc.store_scatter(t_v.at[t], [dts[j], dls[j], bsplat],
                                   g + pv[j])

        pltpu.async_copy(t_v.at[t, :, :, pl.ds(0, BT)],
                         out_hbm.at[l, :, w], osem[t])

    for l in range(NBG - 1):
        fetch_idx(l, l)
    for l in range(NBG - 2):
        start_gather(l, l)

    def step(s, carry):
        for u in range(NBG):
            l = NBG * s + u

            @pl.when(l + NBG - 1 < MAX_LEN)
            def _():
                fetch_idx(l + NBG - 1, (u + NBG - 1) % NBG)

            @pl.when(l + NBG - 2 < MAX_LEN)
            def _():
                start_gather(l + NBG - 2, (u + NBG - 2) % NBG)

            t = u % NBT

            @pl.when(l >= NBT)
            def _():
                wait_out(l - NBT, t)

            compute(l, u, t)
        return carry

    lax.fori_loop(0, MAX_LEN // NBG, step, 0)

    for t in range(NBT):
        wait_out(MAX_LEN - NBT + t, t)


def kernel(inputs, token_table, pos_table):
    idx4 = (inputs.astype(jnp.int32).T
            .reshape(LT, 8, NW, BT)
            .transpose(0, 2, 1, 3))
    mesh = plsc.VectorSubcoreMesh(core_axis_name="c", subcore_axis_name="s")
    run = functools.partial(
        pl.kernel,
        mesh=mesh,
        compiler_params=pltpu.CompilerParams(use_tc_tiling_on_sc=False,
                                             needs_layout_passes=False),
        out_type=jax.ShapeDtypeStruct((MAX_LEN, DT, NW, 8, BT), jnp.float32),
        scratch_types=[
            pltpu.VMEM((NBG, BT), jnp.int32),
            pltpu.VMEM((NBG, BT, EMBED), jnp.float32),
            pltpu.VMEM((NBT, DT, 8, BTP), jnp.float32),
            pltpu.VMEM((MAX_LEN, EMBED), jnp.float32),
        ] + [pltpu.SemaphoreType.DMA] * (2 * NBG + NBT),
    )(_body)
    out = run(idx4, token_table, pos_table)
    return out.transpose(2, 4, 0, 1, 3).reshape(BATCH, MAX_LEN, EMBED)
